# Initial kernel scaffold; baseline (speedup 1.0000x reference)
import functools

import jax
import jax.numpy as jnp
from jax import lax
from jax.experimental import pallas as pl
from jax.experimental.pallas import tpu as pltpu
from jax.experimental.pallas import tpu_sc as plsc

_N = 10000
_E = 320000
_H = 64

_NC = 2
_NS = 16
_NW = _NC * _NS
_EPW = _E // _NW
_CLEN = 125
_NCHUNK = _EPW // _CLEN
_RPT = _N // _NS
_VSL = _H // 16


def _spmm_sc(src3, dst3, ew3, h):
    mesh = plsc.VectorSubcoreMesh(core_axis_name="c", subcore_axis_name="s")

    @functools.partial(
        pl.kernel,
        mesh=mesh,
        out_type=jax.ShapeDtypeStruct((_NC, _N, _H), jnp.float32),
        scratch_types=[
            pltpu.VMEM((_NCHUNK, _CLEN), jnp.int32),
            pltpu.VMEM((_NCHUNK, _CLEN), jnp.int32),
            pltpu.VMEM((_NCHUNK, _CLEN), jnp.float32),
            pltpu.VMEM((_CLEN, _H), jnp.float32),
            pltpu.VMEM_SHARED((_N, _H), jnp.float32),
            pltpu.SemaphoreType.DMA,
        ],
    )
    def spmm(src_hbm, dst_hbm, w_hbm, h_hbm, out_hbm,
             src_v, dst_v, w_v, rows_v, acc_sh, sem):
        c = lax.axis_index("c")
        s = lax.axis_index("s")
        wid = s * _NC + c

        pltpu.sync_copy(src_hbm.at[wid], src_v)
        pltpu.sync_copy(dst_hbm.at[wid], dst_v)
        pltpu.sync_copy(w_hbm.at[wid], w_v)

        def zrow(i, carry):
            for k in range(_VSL):
                rows_v[i, pl.ds(k * 16, 16)] = jnp.zeros((16,), jnp.float32)
            return carry

        lax.fori_loop(0, _CLEN, zrow, 0)
        r0 = s * _RPT
        for t in range(_RPT // _CLEN):
            pltpu.sync_copy(rows_v, acc_sh.at[pl.ds(r0 + t * _CLEN, _CLEN)])
        plsc.subcore_barrier()

        def chunk(j, carry):
            pltpu.async_copy(h_hbm.at[src_v.at[j]], rows_v, sem).wait()

            def edge(i, icarry):
                w_s = w_v[j, i]
                for k in range(_VSL):
                    sl = pl.ds(k * 16, 16)
                    rows_v[i, sl] = rows_v[i, sl] * w_s
                return icarry

            lax.fori_loop(0, _CLEN, edge, 0)
            pltpu.sync_copy(rows_v, acc_sh.at[dst_v.at[j]], add=True)
            return carry

        lax.fori_loop(0, _NCHUNK, chunk, 0)
        plsc.subcore_barrier()

        for t in range(_RPT // _CLEN):
            rr = r0 + t * _CLEN
            pltpu.sync_copy(acc_sh.at[pl.ds(rr, _CLEN)], rows_v)
            pltpu.sync_copy(rows_v, out_hbm.at[c, pl.ds(rr, _CLEN)])

    return spmm(src3, dst3, ew3, h)


def _linear_relu(x, W, b2d):
    def body(x_ref, w_ref, b_ref, o_ref):
        o_ref[...] = jnp.maximum(
            jnp.dot(x_ref[...], w_ref[...],
                    preferred_element_type=jnp.float32) + b_ref[...],
            0.0)

    return pl.pallas_call(
        body,
        out_shape=jax.ShapeDtypeStruct((x.shape[0], W.shape[1]), jnp.float32),
    )(x, W, b2d)


def _sum_linear_relu(p, W, b2d):
    def body(p_ref, w_ref, b_ref, o_ref):
        h = p_ref[0] + p_ref[1]
        o_ref[...] = jnp.maximum(
            jnp.dot(h, w_ref[...], preferred_element_type=jnp.float32)
            + b_ref[...],
            0.0)

    return pl.pallas_call(
        body,
        out_shape=jax.ShapeDtypeStruct((p.shape[1], W.shape[1]), jnp.float32),
    )(p, W, b2d)


def _readout(p, Wf1, bf1_2d, Wf2, bf2_2d):
    def body(p_ref, w1_ref, b1_ref, w2_ref, b2_ref, o_ref):
        h = p_ref[0] + p_ref[1]
        ge = jnp.sum(h, axis=0, keepdims=True) * (1.0 / _N)
        o1 = jnp.maximum(
            jnp.dot(ge, w1_ref[...], preferred_element_type=jnp.float32)
            + b1_ref[...],
            0.0)
        r = jax.nn.sigmoid(
            jnp.dot(o1, w2_ref[...], preferred_element_type=jnp.float32)
            + b2_ref[...])
        o_ref[...] = r

    return pl.pallas_call(
        body,
        out_shape=jax.ShapeDtypeStruct((1, 1), jnp.float32),
    )(p, Wf1, bf1_2d, Wf2, bf2_2d)


def kernel(x, edge_index, edge_weight, W1, b1, W2, b2, W3, b3,
           Wf1, bf1, Wf2, bf2):
    dst3 = edge_index[0].reshape(_NW, _NCHUNK, _CLEN)
    src3 = edge_index[1].reshape(_NW, _NCHUNK, _CLEN)
    ew3 = edge_weight.reshape(_NW, _NCHUNK, _CLEN)

    h = _linear_relu(x, W1, b1.reshape(1, _H))
    p = _spmm_sc(src3, dst3, ew3, h)
    h = _sum_linear_relu(p, W2, b2.reshape(1, _H))
    p = _spmm_sc(src3, dst3, ew3, h)
    h = _sum_linear_relu(p, W3, b3.reshape(1, _H))
    p = _spmm_sc(src3, dst3, ew3, h)
    r = _readout(p, Wf1, bf1.reshape(1, 32), Wf2, bf2.reshape(1, 1))
    return jnp.squeeze(r)


# speedup vs baseline: 4.9185x; 4.9185x over previous
import functools

import jax
import jax.numpy as jnp
from jax import lax
from jax.experimental import pallas as pl
from jax.experimental.pallas import tpu as pltpu
from jax.experimental.pallas import tpu_sc as plsc

_N = 10000
_E = 320000
_H = 64

_NC = 2
_NS = 16
_NW = _NC * _NS
_EPW = _E // _NW
_CLEN = 80
_NCHUNK = _EPW // _CLEN
_RA = 624
_ZC = 208
_TAIL0 = _NS * _RA
_TAILN = _N - _TAIL0
_VSL = _H // 16


def _spmm_sc(src3, dst3, ew3, h):
    mesh = plsc.VectorSubcoreMesh(core_axis_name="c", subcore_axis_name="s")

    @functools.partial(
        pl.kernel,
        mesh=mesh,
        out_type=jax.ShapeDtypeStruct((_NC, _N, _H), jnp.float32),
        scratch_types=[
            pltpu.VMEM((_NCHUNK, _CLEN), jnp.int32),
            pltpu.VMEM((_NCHUNK, _CLEN), jnp.int32),
            pltpu.VMEM((_NCHUNK, _CLEN), jnp.float32),
            pltpu.VMEM((_CLEN, _H), jnp.float32),
            pltpu.VMEM((_ZC, _H), jnp.float32),
            pltpu.VMEM_SHARED((_N, _H), jnp.float32),
            pltpu.SemaphoreType.DMA,
        ],
        compiler_params=pltpu.CompilerParams(use_tc_tiling_on_sc=False),
    )
    def spmm(src_hbm, dst_hbm, w_hbm, h_hbm, out_hbm,
             src_v, dst_v, w_v, rows_v, z_v, acc_sh, sem):
        c = lax.axis_index("c")
        s = lax.axis_index("s")
        wid = s * _NC + c

        pltpu.sync_copy(src_hbm.at[wid], src_v)
        pltpu.sync_copy(dst_hbm.at[wid], dst_v)
        pltpu.sync_copy(w_hbm.at[wid], w_v)

        def zrow(i, carry):
            for k in range(_VSL):
                z_v[i, pl.ds(k * 16, 16)] = jnp.zeros((16,), jnp.float32)
            return carry

        lax.fori_loop(0, _ZC, zrow, 0)
        r0 = s * _RA
        for t in range(_RA // _ZC):
            pltpu.sync_copy(z_v, acc_sh.at[pl.ds(r0 + t * _ZC, _ZC)])

        @pl.when(s == 0)
        def _zero_tail():
            pltpu.sync_copy(z_v.at[pl.ds(0, _TAILN)],
                            acc_sh.at[pl.ds(_TAIL0, _TAILN)])

        plsc.subcore_barrier()

        def chunk(j, carry):
            pltpu.async_copy(h_hbm.at[src_v.at[j]], rows_v, sem).wait()

            def group(g, gcarry):
                w16 = w_v[j, pl.ds(g * 16, 16)]
                base = g * 16
                for e in range(16):
                    w_s = w16[e]
                    for k in range(_VSL):
                        sl = pl.ds(k * 16, 16)
                        rows_v[base + e, sl] = rows_v[base + e, sl] * w_s
                return gcarry

            lax.fori_loop(0, _CLEN // 16, group, 0)
            pltpu.sync_copy(rows_v, acc_sh.at[dst_v.at[j]], add=True)
            return carry

        lax.fori_loop(0, _NCHUNK, chunk, 0)
        plsc.subcore_barrier()

        for t in range(_RA // _ZC):
            rr = r0 + t * _ZC
            pltpu.sync_copy(acc_sh.at[pl.ds(rr, _ZC)], z_v)
            pltpu.sync_copy(z_v, out_hbm.at[c, pl.ds(rr, _ZC)])

        @pl.when(s == 0)
        def _out_tail():
            pltpu.sync_copy(acc_sh.at[pl.ds(_TAIL0, _TAILN)],
                            z_v.at[pl.ds(0, _TAILN)])
            pltpu.sync_copy(z_v.at[pl.ds(0, _TAILN)],
                            out_hbm.at[c, pl.ds(_TAIL0, _TAILN)])

    return spmm(src3, dst3, ew3, h)


def _linear_relu(x, W, b2d):
    def body(x_ref, w_ref, b_ref, o_ref):
        o_ref[...] = jnp.maximum(
            jnp.dot(x_ref[...], w_ref[...],
                    preferred_element_type=jnp.float32) + b_ref[...],
            0.0)

    return pl.pallas_call(
        body,
        out_shape=jax.ShapeDtypeStruct((x.shape[0], W.shape[1]), jnp.float32),
    )(x, W, b2d)


def _sum_linear_relu(p, W, b2d):
    def body(p_ref, w_ref, b_ref, o_ref):
        h = p_ref[0] + p_ref[1]
        o_ref[...] = jnp.maximum(
            jnp.dot(h, w_ref[...], preferred_element_type=jnp.float32)
            + b_ref[...],
            0.0)

    return pl.pallas_call(
        body,
        out_shape=jax.ShapeDtypeStruct((p.shape[1], W.shape[1]), jnp.float32),
    )(p, W, b2d)


def _readout(p, Wf1, bf1_2d, Wf2, bf2_2d):
    def body(p_ref, w1_ref, b1_ref, w2_ref, b2_ref, o_ref):
        def chunk_sum(t, acc):
            c = p_ref[0, pl.ds(t * 80, 80), :] + p_ref[1, pl.ds(t * 80, 80), :]
            return acc + jnp.sum(c, axis=0, keepdims=True)

        ge = lax.fori_loop(
            0, _N // 80, chunk_sum,
            jnp.zeros((1, _H), jnp.float32)) * (1.0 / _N)
        o1 = jnp.maximum(
            jnp.dot(ge, w1_ref[...], preferred_element_type=jnp.float32)
            + b1_ref[...],
            0.0)
        r = jax.nn.sigmoid(
            jnp.dot(o1, w2_ref[...], preferred_element_type=jnp.float32)
            + b2_ref[...])
        o_ref[...] = r

    return pl.pallas_call(
        body,
        out_shape=jax.ShapeDtypeStruct((1, 1), jnp.float32),
    )(p, Wf1, bf1_2d, Wf2, bf2_2d)


def kernel(x, edge_index, edge_weight, W1, b1, W2, b2, W3, b3,
           Wf1, bf1, Wf2, bf2):
    dst3 = edge_index[0].reshape(_NW, _NCHUNK, _CLEN)
    src3 = edge_index[1].reshape(_NW, _NCHUNK, _CLEN)
    ew3 = edge_weight.reshape(_NW, _NCHUNK, _CLEN)

    h = _linear_relu(x, W1, b1.reshape(1, _H))
    p = _spmm_sc(src3, dst3, ew3, h)
    h = _sum_linear_relu(p, W2, b2.reshape(1, _H))
    p = _spmm_sc(src3, dst3, ew3, h)
    h = _sum_linear_relu(p, W3, b3.reshape(1, _H))
    p = _spmm_sc(src3, dst3, ew3, h)
    r = _readout(p, Wf1, bf1.reshape(1, 32), Wf2, bf2.reshape(1, 1))
    return jnp.squeeze(r)


# speedup vs baseline: 4.9233x; 1.0010x over previous
import functools

import jax
import jax.numpy as jnp
from jax import lax
from jax.experimental import pallas as pl
from jax.experimental.pallas import tpu as pltpu
from jax.experimental.pallas import tpu_sc as plsc

_N = 10000
_E = 320000
_H = 64

_NC = 2
_NS = 16
_NW = _NC * _NS
_EPW = _E // _NW
_CLEN = 80
_NCHUNK = _EPW // _CLEN
_RA = 624
_ZC = 208
_TAIL0 = _NS * _RA
_TAILN = _N - _TAIL0
_VSL = _H // 16


def _spmm_sc(src3, dst3, ew3, h):
    mesh = plsc.VectorSubcoreMesh(core_axis_name="c", subcore_axis_name="s")

    @functools.partial(
        pl.kernel,
        mesh=mesh,
        out_type=jax.ShapeDtypeStruct((_NC, _N, _H), jnp.float32),
        scratch_types=[
            pltpu.VMEM((_NCHUNK, _CLEN), jnp.int32),
            pltpu.VMEM((_NCHUNK, _CLEN), jnp.int32),
            pltpu.VMEM((_NCHUNK, _CLEN), jnp.float32),
            pltpu.VMEM((_CLEN, _H), jnp.float32),
            pltpu.VMEM((_ZC, _H), jnp.float32),
            pltpu.VMEM_SHARED((_N, _H), jnp.float32),
            pltpu.SemaphoreType.DMA,
        ],
        compiler_params=pltpu.CompilerParams(use_tc_tiling_on_sc=False),
    )
    def spmm(src_hbm, dst_hbm, w_hbm, h_hbm, out_hbm,
             src_v, dst_v, w_v, rows_v, z_v, acc_sh, sem):
        c = lax.axis_index("c")
        s = lax.axis_index("s")
        wid = s * _NC + c

        pltpu.sync_copy(src_hbm.at[wid], src_v)
        pltpu.sync_copy(dst_hbm.at[wid], dst_v)
        pltpu.sync_copy(w_hbm.at[wid], w_v)

        def zrow(i, carry):
            for k in range(_VSL):
                z_v[i, pl.ds(k * 16, 16)] = jnp.zeros((16,), jnp.float32)
            return carry

        lax.fori_loop(0, _ZC, zrow, 0)
        r0 = s * _RA
        for t in range(_RA // _ZC):
            pltpu.sync_copy(z_v, acc_sh.at[pl.ds(r0 + t * _ZC, _ZC)])

        @pl.when(s == 0)
        def _zero_tail():
            pltpu.sync_copy(z_v.at[pl.ds(0, _TAILN)],
                            acc_sh.at[pl.ds(_TAIL0, _TAILN)])

        plsc.subcore_barrier()

        def chunk(j, carry):
            pltpu.async_copy(h_hbm.at[src_v.at[j]], rows_v, sem).wait()

            def group(g, gcarry):
                w16 = w_v[j, pl.ds(g * 16, 16)]
                base = g * 16
                for e in range(16):
                    w_s = w16[e]
                    for k in range(_VSL):
                        sl = pl.ds(k * 16, 16)
                        rows_v[base + e, sl] = rows_v[base + e, sl] * w_s
                return gcarry

            lax.fori_loop(0, _CLEN // 16, group, 0)
            pltpu.sync_copy(rows_v, acc_sh.at[dst_v.at[j]], add=True)
            return carry

        lax.fori_loop(0, _NCHUNK, chunk, 0)
        plsc.subcore_barrier()

        for t in range(_RA // _ZC):
            rr = r0 + t * _ZC
            pltpu.sync_copy(acc_sh.at[pl.ds(rr, _ZC)], z_v)
            pltpu.sync_copy(z_v, out_hbm.at[c, pl.ds(rr, _ZC)])

        @pl.when(s == 0)
        def _out_tail():
            pltpu.sync_copy(acc_sh.at[pl.ds(_TAIL0, _TAILN)],
                            z_v.at[pl.ds(0, _TAILN)])
            pltpu.sync_copy(z_v.at[pl.ds(0, _TAILN)],
                            out_hbm.at[c, pl.ds(_TAIL0, _TAILN)])

    return spmm(src3, dst3, ew3, h)


def _linear_relu(x, W, b2d):
    def body(x_ref, w_ref, b_ref, o_ref):
        o_ref[...] = jnp.maximum(
            jnp.dot(x_ref[...], w_ref[...],
                    preferred_element_type=jnp.float32) + b_ref[...],
            0.0)

    return pl.pallas_call(
        body,
        out_shape=jax.ShapeDtypeStruct((x.shape[0], W.shape[1]), jnp.float32),
    )(x, W, b2d)


def _sum_linear_relu(p, W, b2d):
    def body(p_ref, w_ref, b_ref, o_ref):
        h = p_ref[0] + p_ref[1]
        o_ref[...] = jnp.maximum(
            jnp.dot(h, w_ref[...], preferred_element_type=jnp.float32)
            + b_ref[...],
            0.0)

    return pl.pallas_call(
        body,
        out_shape=jax.ShapeDtypeStruct((p.shape[1], W.shape[1]), jnp.float32),
    )(p, W, b2d)


def _readout(p, Wf1, bf1_2d, Wf2T, bf2_2d):
    def body(p_ref, w1_ref, b1_ref, w2t_ref, b2_ref, o_ref):
        def chunk_sum(t, acc):
            c = p_ref[0, pl.ds(t * 80, 80), :] + p_ref[1, pl.ds(t * 80, 80), :]
            return acc + jnp.sum(c, axis=0, keepdims=True)

        ge = lax.fori_loop(
            0, _N // 80, chunk_sum,
            jnp.zeros((1, _H), jnp.float32)) * (1.0 / _N)
        o1 = jnp.maximum(
            jnp.dot(ge.astype(jnp.bfloat16),
                    w1_ref[...].astype(jnp.bfloat16),
                    preferred_element_type=jnp.float32) + b1_ref[...],
            0.0)
        pre = jnp.sum(o1 * w2t_ref[...], axis=1, keepdims=True) + b2_ref[...]
        o_ref[...] = jax.nn.sigmoid(pre)

    return pl.pallas_call(
        body,
        out_shape=jax.ShapeDtypeStruct((1, 1), jnp.float32),
    )(p, Wf1, bf1_2d, Wf2T, bf2_2d)


def kernel(x, edge_index, edge_weight, W1, b1, W2, b2, W3, b3,
           Wf1, bf1, Wf2, bf2):
    dst3 = edge_index[0].reshape(_NW, _NCHUNK, _CLEN)
    src3 = edge_index[1].reshape(_NW, _NCHUNK, _CLEN)
    ew3 = edge_weight.reshape(_NW, _NCHUNK, _CLEN)

    h = _linear_relu(x, W1, b1.reshape(1, _H))
    p = _spmm_sc(src3, dst3, ew3, h)
    h = _sum_linear_relu(p, W2, b2.reshape(1, _H))
    p = _spmm_sc(src3, dst3, ew3, h)
    h = _sum_linear_relu(p, W3, b3.reshape(1, _H))
    p = _spmm_sc(src3, dst3, ew3, h)
    r = _readout(p, Wf1, bf1.reshape(1, 32), Wf2.reshape(1, 32),
                 bf2.reshape(1, 1))
    return jnp.squeeze(r)


# speedup vs baseline: 6.9675x; 1.4152x over previous
import functools

import jax
import jax.numpy as jnp
from jax import lax
from jax.experimental import pallas as pl
from jax.experimental.pallas import tpu as pltpu
from jax.experimental.pallas import tpu_sc as plsc

_N = 10000
_E = 320000
_H = 64

_NC = 2
_NS = 16
_NW = _NC * _NS
_EPW = _E // _NW
_CLEN = 80
_NCHUNK = _EPW // _CLEN
_RA = 624
_ZC = 208
_TAIL0 = _NS * _RA
_TAILN = _N - _TAIL0
_VSL = _H // 16


def _spmm_sc(src3, dst3, ew3, h):
    mesh = plsc.VectorSubcoreMesh(core_axis_name="c", subcore_axis_name="s")

    @functools.partial(
        pl.kernel,
        mesh=mesh,
        out_type=jax.ShapeDtypeStruct((_NC, _N, _H), jnp.float32),
        scratch_types=[
            pltpu.VMEM((_NCHUNK, _CLEN), jnp.int32),
            pltpu.VMEM((_NCHUNK, _CLEN), jnp.int32),
            pltpu.VMEM((_NCHUNK, _CLEN), jnp.float32),
            pltpu.VMEM((_CLEN, _H), jnp.float32),
            pltpu.VMEM((_CLEN, _H), jnp.float32),
            pltpu.VMEM((_ZC, _H), jnp.float32),
            pltpu.VMEM_SHARED((_N, _H), jnp.float32),
            pltpu.SemaphoreType.DMA,
            pltpu.SemaphoreType.DMA,
        ],
        compiler_params=pltpu.CompilerParams(use_tc_tiling_on_sc=False),
    )
    def spmm(src_hbm, dst_hbm, w_hbm, h_hbm, out_hbm,
             src_v, dst_v, w_v, rows_a, rows_b, z_v, acc_sh, sem_a, sem_b):
        c = lax.axis_index("c")
        s = lax.axis_index("s")
        wid = s * _NC + c

        pltpu.sync_copy(src_hbm.at[wid], src_v)
        pltpu.sync_copy(dst_hbm.at[wid], dst_v)
        pltpu.sync_copy(w_hbm.at[wid], w_v)

        def zrow(i, carry):
            for k in range(_VSL):
                z_v[i, pl.ds(k * 16, 16)] = jnp.zeros((16,), jnp.float32)
            return carry

        lax.fori_loop(0, _ZC, zrow, 0)
        r0 = s * _RA
        for t in range(_RA // _ZC):
            pltpu.sync_copy(z_v, acc_sh.at[pl.ds(r0 + t * _ZC, _ZC)])

        @pl.when(s == 0)
        def _zero_tail():
            pltpu.sync_copy(z_v.at[pl.ds(0, _TAILN)],
                            acc_sh.at[pl.ds(_TAIL0, _TAILN)])

        plsc.subcore_barrier()

        def scale(jj, buf):
            def group(g, gcarry):
                w16 = w_v[jj, pl.ds(g * 16, 16)]
                base = g * 16
                for e in range(16):
                    w_s = w16[e]
                    for k in range(_VSL):
                        sl = pl.ds(k * 16, 16)
                        buf[base + e, sl] = buf[base + e, sl] * w_s
                return gcarry

            lax.fori_loop(0, _CLEN // 16, group, 0)

        npair = _NCHUNK // 2
        pltpu.async_copy(h_hbm.at[src_v.at[0]], rows_a, sem_a)
        pltpu.async_copy(h_hbm.at[src_v.at[1]], rows_b, sem_b)

        def pair(j, carry):
            j0 = 2 * j
            pltpu.make_async_copy(h_hbm.at[src_v.at[j0]], rows_a, sem_a).wait()
            scale(j0, rows_a)
            pltpu.sync_copy(rows_a, acc_sh.at[dst_v.at[j0]], add=True)
            pltpu.async_copy(h_hbm.at[src_v.at[j0 + 2]], rows_a, sem_a)

            pltpu.make_async_copy(h_hbm.at[src_v.at[j0 + 1]], rows_b,
                                  sem_b).wait()
            scale(j0 + 1, rows_b)
            pltpu.sync_copy(rows_b, acc_sh.at[dst_v.at[j0 + 1]], add=True)

            @pl.when(j < npair - 1)
            def _start_b():
                pltpu.async_copy(h_hbm.at[src_v.at[j0 + 3]], rows_b, sem_b)

            return carry

        lax.fori_loop(0, npair, pair, 0)
        last = _NCHUNK - 1
        pltpu.make_async_copy(h_hbm.at[src_v.at[last]], rows_a, sem_a).wait()
        scale(last, rows_a)
        pltpu.sync_copy(rows_a, acc_sh.at[dst_v.at[last]], add=True)
        plsc.subcore_barrier()

        for t in range(_RA // _ZC):
            rr = r0 + t * _ZC
            pltpu.sync_copy(acc_sh.at[pl.ds(rr, _ZC)], z_v)
            pltpu.sync_copy(z_v, out_hbm.at[c, pl.ds(rr, _ZC)])

        @pl.when(s == 0)
        def _out_tail():
            pltpu.sync_copy(acc_sh.at[pl.ds(_TAIL0, _TAILN)],
                            z_v.at[pl.ds(0, _TAILN)])
            pltpu.sync_copy(z_v.at[pl.ds(0, _TAILN)],
                            out_hbm.at[c, pl.ds(_TAIL0, _TAILN)])

    return spmm(src3, dst3, ew3, h)


def _linear_relu(x, W, b2d):
    def body(x_ref, w_ref, b_ref, o_ref):
        o_ref[...] = jnp.maximum(
            jnp.dot(x_ref[...], w_ref[...],
                    preferred_element_type=jnp.float32) + b_ref[...],
            0.0)

    return pl.pallas_call(
        body,
        out_shape=jax.ShapeDtypeStruct((x.shape[0], W.shape[1]), jnp.float32),
    )(x, W, b2d)


def _sum_linear_relu(p, W, b2d):
    def body(p_ref, w_ref, b_ref, o_ref):
        h = p_ref[0] + p_ref[1]
        o_ref[...] = jnp.maximum(
            jnp.dot(h, w_ref[...], preferred_element_type=jnp.float32)
            + b_ref[...],
            0.0)

    return pl.pallas_call(
        body,
        out_shape=jax.ShapeDtypeStruct((p.shape[1], W.shape[1]), jnp.float32),
    )(p, W, b2d)


def _readout(p, Wf1, bf1_2d, Wf2T, bf2_2d):
    def body(p_ref, w1_ref, b1_ref, w2t_ref, b2_ref, o_ref):
        def chunk_sum(t, acc):
            c = p_ref[0, pl.ds(t * 80, 80), :] + p_ref[1, pl.ds(t * 80, 80), :]
            return acc + jnp.sum(c, axis=0, keepdims=True)

        ge = lax.fori_loop(
            0, _N // 80, chunk_sum,
            jnp.zeros((1, _H), jnp.float32)) * (1.0 / _N)
        o1 = jnp.maximum(
            jnp.dot(ge.astype(jnp.bfloat16),
                    w1_ref[...].astype(jnp.bfloat16),
                    preferred_element_type=jnp.float32) + b1_ref[...],
            0.0)
        pre = jnp.sum(o1 * w2t_ref[...], axis=1, keepdims=True) + b2_ref[...]
        o_ref[...] = jax.nn.sigmoid(pre)

    return pl.pallas_call(
        body,
        out_shape=jax.ShapeDtypeStruct((1, 1), jnp.float32),
    )(p, Wf1, bf1_2d, Wf2T, bf2_2d)


def kernel(x, edge_index, edge_weight, W1, b1, W2, b2, W3, b3,
           Wf1, bf1, Wf2, bf2):
    dst3 = edge_index[0].reshape(_NW, _NCHUNK, _CLEN)
    src3 = edge_index[1].reshape(_NW, _NCHUNK, _CLEN)
    ew3 = edge_weight.reshape(_NW, _NCHUNK, _CLEN)

    h = _linear_relu(x, W1, b1.reshape(1, _H))
    p = _spmm_sc(src3, dst3, ew3, h)
    h = _sum_linear_relu(p, W2, b2.reshape(1, _H))
    p = _spmm_sc(src3, dst3, ew3, h)
    h = _sum_linear_relu(p, W3, b3.reshape(1, _H))
    p = _spmm_sc(src3, dst3, ew3, h)
    r = _readout(p, Wf1, bf1.reshape(1, 32), Wf2.reshape(1, 32),
                 bf2.reshape(1, 1))
    return jnp.squeeze(r)
